# baseline (speedup 1.0000x reference)
import jax
import jax.numpy as jnp
from jax import lax
from jax.experimental import pallas as pl
from jax.experimental.pallas import tpu as pltpu
from jax.experimental.pallas import tpu_sc as plsc

NUM_EMB = 100000
EMB_DIM = 300
DPAD = 304
BATCH = 1024
SEQ = 50
B = BATCH * SEQ

_info = plsc.get_sparse_core_info()
NC = _info.num_cores
NS = _info.num_subcores
NW = NC * NS
BPW = B // NW
CHUNK = 80
NCHUNK = BPW // CHUNK


def _gather_kernel(idx_hbm, table_hbm, out_hbm, idx_v, rows_v, sem):
    wid = lax.axis_index("s") * NC + lax.axis_index("c")
    base = wid * BPW
    pltpu.sync_copy(idx_hbm.at[wid], idx_v)
    for j in range(NCHUNK):
        pltpu.async_copy(table_hbm.at[idx_v.at[j]], rows_v, sem).wait()
        pltpu.sync_copy(rows_v, out_hbm.at[pl.ds(base + j * CHUNK, CHUNK)])


@jax.jit
def kernel(x, table):
    idx = x.reshape(NW, NCHUNK, CHUNK).astype(jnp.int32)
    table_p = jnp.pad(table, ((0, 0), (0, DPAD - EMB_DIM)))
    mesh = plsc.VectorSubcoreMesh(core_axis_name="c", subcore_axis_name="s")
    out = pl.kernel(
        _gather_kernel,
        out_type=jax.ShapeDtypeStruct((B, DPAD), jnp.float32),
        mesh=mesh,
        scratch_types=[
            pltpu.VMEM((NCHUNK, CHUNK), jnp.int32),
            pltpu.VMEM((CHUNK, DPAD), jnp.float32),
            pltpu.SemaphoreType.DMA,
        ],
        compiler_params=pltpu.CompilerParams(use_tc_tiling_on_sc=False),
    )(idx, table_p)
    return out[:, :EMB_DIM].reshape(BATCH, SEQ, EMB_DIM)


# speedup vs baseline: 1.3006x; 1.3006x over previous
import jax
import jax.numpy as jnp
from jax import lax
from jax.experimental import pallas as pl
from jax.experimental.pallas import tpu as pltpu
from jax.experimental.pallas import tpu_sc as plsc

NUM_EMB = 100000
EMB_DIM = 300
DPAD = 384
BATCH = 1024
SEQ = 50
B = BATCH * SEQ

_info = plsc.get_sparse_core_info()
NC = _info.num_cores
NS = _info.num_subcores
NW = NC * NS
BPW = B // NW
CHUNK = 80
NCHUNK = BPW // CHUNK


def _gather_kernel(idx_hbm, table_hbm, out_hbm, idx_v, rows_v, sem):
    wid = lax.axis_index("s") * NC + lax.axis_index("c")
    base = wid * BPW
    pltpu.sync_copy(idx_hbm.at[wid], idx_v)
    for j in range(NCHUNK):
        pltpu.async_copy(table_hbm.at[idx_v.at[j]], rows_v, sem).wait()
        pltpu.sync_copy(rows_v, out_hbm.at[pl.ds(base + j * CHUNK, CHUNK)])


@jax.jit
def kernel(x, table):
    idx = x.reshape(NW, NCHUNK, CHUNK).astype(jnp.int32)
    table_p = jnp.pad(table, ((0, 0), (0, DPAD - EMB_DIM)))
    mesh = plsc.VectorSubcoreMesh(core_axis_name="c", subcore_axis_name="s")
    out = pl.kernel(
        _gather_kernel,
        out_type=jax.ShapeDtypeStruct((B, DPAD), jnp.float32),
        mesh=mesh,
        scratch_types=[
            pltpu.VMEM((NCHUNK, CHUNK), jnp.int32),
            pltpu.VMEM((CHUNK, DPAD), jnp.float32),
            pltpu.SemaphoreType.DMA,
        ],
    )(idx, table_p)
    return out[:, :EMB_DIM].reshape(BATCH, SEQ, EMB_DIM)


# speedup vs baseline: 1.6801x; 1.2918x over previous
import jax
import jax.numpy as jnp
from jax import lax
from jax.experimental import pallas as pl
from jax.experimental.pallas import tpu as pltpu
from jax.experimental.pallas import tpu_sc as plsc

NUM_EMB = 100000
EMB_DIM = 300
BATCH = 1024
SEQ = 50
B = BATCH * SEQ

_info = plsc.get_sparse_core_info()
NC = _info.num_cores
NS = _info.num_subcores
NW = NC * NS
L = _info.num_lanes
ROWS_PER = -(-EMB_DIM // NW)
JC = 12800
NJC = B // JC


def _gather_kernel(idx_hbm, tab_hbm, out_hbm, idx_v, row_v, out_v):
    wid = lax.axis_index("s") * NC + lax.axis_index("c")
    for t in range(ROWS_PER):
        r = wid + t * NW

        @pl.when(r < EMB_DIM)
        def _():
            pltpu.sync_copy(tab_hbm.at[r], row_v)
            for c in range(NJC):
                pltpu.sync_copy(idx_hbm.at[pl.ds(c * JC, JC)], idx_v)

                def body(k, carry):
                    iv = idx_v[pl.ds(k * L, L)]
                    out_v[pl.ds(k * L, L)] = plsc.load_gather(row_v, [iv])
                    return carry

                lax.fori_loop(0, JC // L, body, 0)
                pltpu.sync_copy(out_v, out_hbm.at[r, pl.ds(c * JC, JC)])


def kernel(x, table):
    idx = x.reshape(B).astype(jnp.int32)
    tab_t = table.T
    mesh = plsc.VectorSubcoreMesh(core_axis_name="c", subcore_axis_name="s")
    out_t = pl.kernel(
        _gather_kernel,
        out_type=jax.ShapeDtypeStruct((EMB_DIM, B), jnp.float32),
        mesh=mesh,
        scratch_types=[
            pltpu.VMEM((JC,), jnp.int32),
            pltpu.VMEM((NUM_EMB,), jnp.float32),
            pltpu.VMEM((JC,), jnp.float32),
        ],
        compiler_params=pltpu.CompilerParams(needs_layout_passes=False),
    )(idx, tab_t)
    return out_t.T.reshape(BATCH, SEQ, EMB_DIM)


# speedup vs baseline: 3.2247x; 1.9193x over previous
import jax
import jax.numpy as jnp
from jax import lax
from jax.experimental import pallas as pl
from jax.experimental.pallas import tpu as pltpu
from jax.experimental.pallas import tpu_sc as plsc

NUM_EMB = 100000
EMB_DIM = 300
BATCH = 1024
SEQ = 50
B = BATCH * SEQ

_info = plsc.get_sparse_core_info()
NC = _info.num_cores
NS = _info.num_subcores
NW = NC * NS
L = _info.num_lanes
ROWS_PER = -(-EMB_DIM // NW)
SC_PER_CHUNK = 10
JC = SC_PER_CHUNK * BATCH
NJC = B // JC
UNROLL = 8


def _gather_kernel(idx_hbm, tab_hbm, out_hbm, idx_v, row_v, out_v):
    wid = lax.axis_index("s") * NC + lax.axis_index("c")
    for t in range(ROWS_PER):
        r = wid + t * NW

        @pl.when(r < EMB_DIM)
        def _():
            pltpu.sync_copy(tab_hbm.at[r], row_v)
            for c in range(NJC):
                pltpu.sync_copy(idx_hbm.at[pl.ds(c * JC, JC)], idx_v)

                def body(k, carry):
                    base = k * (L * UNROLL)
                    for u in range(UNROLL):
                        o = base + u * L
                        iv = idx_v[pl.ds(o, L)]
                        out_v[pl.ds(o, L)] = plsc.load_gather(row_v, [iv])
                    return carry

                lax.fori_loop(0, JC // (L * UNROLL), body, 0)
                for si in range(SC_PER_CHUNK):
                    pltpu.sync_copy(
                        out_v.at[pl.ds(si * BATCH, BATCH)],
                        out_hbm.at[c * SC_PER_CHUNK + si, r],
                    )


def kernel(x, table):
    idx = x.T.reshape(B).astype(jnp.int32)
    tab_t = table.T
    mesh = plsc.VectorSubcoreMesh(core_axis_name="c", subcore_axis_name="s")
    out_sdb = pl.kernel(
        _gather_kernel,
        out_type=jax.ShapeDtypeStruct((SEQ, EMB_DIM, BATCH), jnp.float32),
        mesh=mesh,
        scratch_types=[
            pltpu.VMEM((JC,), jnp.int32),
            pltpu.VMEM((NUM_EMB,), jnp.float32),
            pltpu.VMEM((JC,), jnp.float32),
        ],
        compiler_params=pltpu.CompilerParams(needs_layout_passes=False),
    )(idx, tab_t)
    return out_sdb.transpose(2, 0, 1)
